# baseline (speedup 1.0000x reference)
import jax
import jax.numpy as jnp
from jax.experimental import pallas as pl
from jax.experimental.pallas import tpu as pltpu

_VMEM_LIMIT = 56 * 1024 * 1024


def _fused_body(x_ref, w_ref, adj_ref, b_ref, o_ref):
    k = pl.program_id(1)
    sup = jnp.dot(
        x_ref[...].astype(jnp.bfloat16),
        w_ref[...].astype(jnp.bfloat16),
        preferred_element_type=jnp.float32,
    ).astype(jnp.bfloat16)
    part = jnp.dot(
        adj_ref[...].astype(jnp.bfloat16),
        sup,
        preferred_element_type=jnp.float32,
    )

    @pl.when(k == 0)
    def _():
        o_ref[...] = part + b_ref[...]

    @pl.when(k > 0)
    def _():
        o_ref[...] += part


def kernel(x, w, adj, bias):
    n, in_f = x.shape
    out_f = w.shape[1]

    x = x.astype(jnp.float32)
    w = w.astype(jnp.float32)
    adj = adj.astype(jnp.float32)
    bias2d = bias.astype(jnp.float32).reshape(1, out_f)

    num_cores = 2 if n % 2 == 0 else 1
    rows = n // num_cores
    bk = min(n, 512)
    num_k = pl.cdiv(n, bk)

    out = pl.pallas_call(
        _fused_body,
        out_shape=jax.ShapeDtypeStruct((n, out_f), jnp.float32),
        grid=(num_cores, num_k),
        in_specs=[
            pl.BlockSpec((bk, in_f), lambda i, k: (k, 0)),
            pl.BlockSpec((in_f, out_f), lambda i, k: (0, 0),
                         pipeline_mode=pl.Buffered(1)),
            pl.BlockSpec((rows, bk), lambda i, k: (i, k)),
            pl.BlockSpec((1, out_f), lambda i, k: (0, 0),
                         pipeline_mode=pl.Buffered(1)),
        ],
        out_specs=pl.BlockSpec((rows, out_f), lambda i, k: (i, 0)),
        compiler_params=pltpu.CompilerParams(
            dimension_semantics=("parallel", "arbitrary"),
            vmem_limit_bytes=_VMEM_LIMIT,
        ),
    )(x, w, adj, bias2d)

    return out


# speedup vs baseline: 1.1740x; 1.1740x over previous
import jax
import jax.numpy as jnp
from jax.experimental import pallas as pl
from jax.experimental.pallas import tpu as pltpu

_VMEM_LIMIT = 56 * 1024 * 1024


def _fused_body(x_ref, w_ref, adj_ref, b_ref, o_ref):
    k = pl.program_id(1)
    sup = jnp.dot(
        x_ref[...].astype(jnp.bfloat16),
        w_ref[...].astype(jnp.bfloat16),
        preferred_element_type=jnp.float32,
    ).astype(jnp.bfloat16)
    part = jnp.dot(
        adj_ref[...].astype(jnp.bfloat16),
        sup,
        preferred_element_type=jnp.float32,
    )

    @pl.when(k == 0)
    def _():
        o_ref[...] = part + b_ref[...]

    @pl.when(k > 0)
    def _():
        o_ref[...] += part


def kernel(x, w, adj, bias):
    n, in_f = x.shape
    out_f = w.shape[1]

    x = x.astype(jnp.float32)
    w = w.astype(jnp.float32)
    adj = adj.astype(jnp.float32)
    bias2d = bias.astype(jnp.float32).reshape(1, out_f)

    num_cores = 2 if n % 2 == 0 else 1
    rows = n // num_cores
    bk = min(n, 1024)
    num_k = pl.cdiv(n, bk)

    out = pl.pallas_call(
        _fused_body,
        out_shape=jax.ShapeDtypeStruct((n, out_f), jnp.float32),
        grid=(num_cores, num_k),
        in_specs=[
            pl.BlockSpec((bk, in_f), lambda i, k: (k, 0)),
            pl.BlockSpec((in_f, out_f), lambda i, k: (0, 0),
                         pipeline_mode=pl.Buffered(1)),
            pl.BlockSpec((rows, bk), lambda i, k: (i, k)),
            pl.BlockSpec((1, out_f), lambda i, k: (0, 0),
                         pipeline_mode=pl.Buffered(1)),
        ],
        out_specs=pl.BlockSpec((rows, out_f), lambda i, k: (i, 0)),
        compiler_params=pltpu.CompilerParams(
            dimension_semantics=("parallel", "arbitrary"),
            vmem_limit_bytes=_VMEM_LIMIT,
        ),
    )(x, w, adj, bias2d)

    return out


# speedup vs baseline: 1.2273x; 1.0454x over previous
import jax
import jax.numpy as jnp
from jax.experimental import pallas as pl
from jax.experimental.pallas import tpu as pltpu

_VMEM_LIMIT = 56 * 1024 * 1024


def _fused_body(x_ref, w_ref, adj_ref, b_ref, o_ref, sup_ref):
    @pl.when(pl.program_id(1) == 0)
    def _():
        sup_ref[...] = jnp.dot(
            x_ref[...].astype(jnp.bfloat16),
            w_ref[...].astype(jnp.bfloat16),
            preferred_element_type=jnp.float32,
        ).astype(jnp.bfloat16)

    o_ref[...] = (
        jnp.dot(
            adj_ref[...].astype(jnp.bfloat16),
            sup_ref[...],
            preferred_element_type=jnp.float32,
        )
        + b_ref[...]
    )


def kernel(x, w, adj, bias):
    n, in_f = x.shape
    out_f = w.shape[1]

    x = x.astype(jnp.float32)
    w = w.astype(jnp.float32)
    adj = adj.astype(jnp.float32)
    bias2d = bias.astype(jnp.float32).reshape(1, out_f)

    br = min(n, 512)
    num_tiles = pl.cdiv(n, br)
    num_cores = 2 if num_tiles % 2 == 0 else 1
    tiles_per_core = num_tiles // num_cores

    out = pl.pallas_call(
        _fused_body,
        out_shape=jax.ShapeDtypeStruct((n, out_f), jnp.float32),
        grid=(num_cores, tiles_per_core),
        in_specs=[
            pl.BlockSpec((n, in_f), lambda i, k: (0, 0),
                         pipeline_mode=pl.Buffered(1)),
            pl.BlockSpec((in_f, out_f), lambda i, k: (0, 0),
                         pipeline_mode=pl.Buffered(1)),
            pl.BlockSpec((br, n),
                         lambda i, k, t=tiles_per_core: (i * t + k, 0)),
            pl.BlockSpec((1, out_f), lambda i, k: (0, 0),
                         pipeline_mode=pl.Buffered(1)),
        ],
        out_specs=pl.BlockSpec((br, out_f),
                               lambda i, k, t=tiles_per_core: (i * t + k, 0)),
        scratch_shapes=[pltpu.VMEM((n, out_f), jnp.bfloat16)],
        compiler_params=pltpu.CompilerParams(
            dimension_semantics=("parallel", "arbitrary"),
            vmem_limit_bytes=_VMEM_LIMIT,
        ),
    )(x, w, adj, bias2d)

    return out
